# baseline (speedup 1.0000x reference)
import functools

import jax
import jax.numpy as jnp
from jax import lax
from jax.experimental import pallas as pl
from jax.experimental.pallas import tpu as pltpu
from jax.experimental.pallas import tpu_sc as plsc

_DIM_A = 32
_DIM_B = 64
_BATCH, _HIST = 16384, 50
_N = _BATCH * _HIST
_NC, _NS = 2, 16
_NW = _NC * _NS
_PER_W = _N // _NW
_CHUNK_A = 512
_CHUNK_B = 256
_NBUF = 5
_LOOK = 2

_mesh = plsc.VectorSubcoreMesh(core_axis_name="c", subcore_axis_name="s")


@functools.partial(
    pl.kernel,
    mesh=_mesh,
    out_type=[
        jax.ShapeDtypeStruct((_N, _DIM_A), jnp.float32),
        jax.ShapeDtypeStruct((_N, _DIM_B), jnp.float32),
    ],
    scratch_types=[
        pltpu.VMEM((_PER_W,), jnp.int32),
        pltpu.SemaphoreType.DMA((_NBUF,)),
        pltpu.SemaphoreType.DMA((_NBUF,)),
    ],
    compiler_params=pltpu.CompilerParams(use_tc_tiling_on_sc=False),
)
def _gather_kernel(idx_a_hbm, idx_b_hbm, table_a_hbm, table_b_hbm,
                   out_a_hbm, out_b_hbm, idx_v, sem_g, sem_w):
    wid = lax.axis_index("s") * _NC + lax.axis_index("c")
    base = wid * _PER_W

    def phase(idx_hbm, table_hbm, out_hbm, chunk):
        nch = _PER_W // chunk
        pltpu.sync_copy(idx_hbm.at[pl.ds(base, _PER_W)], idx_v)

        def run(rows):
            def start_gather(i, slot):
                pltpu.make_async_copy(
                    table_hbm.at[idx_v.at[pl.ds(i * chunk, chunk)]],
                    rows.at[slot], sem_g.at[slot]).start()

            def wait_gather(slot):
                pltpu.make_async_copy(
                    table_hbm.at[idx_v.at[pl.ds(0, chunk)]],
                    rows.at[slot], sem_g.at[slot]).wait()

            def start_write(i, slot):
                pltpu.make_async_copy(
                    rows.at[slot],
                    out_hbm.at[pl.ds(base + i * chunk, chunk)],
                    sem_w.at[slot]).start()

            def wait_write(slot):
                pltpu.make_async_copy(
                    rows.at[slot],
                    out_hbm.at[pl.ds(base, chunk)],
                    sem_w.at[slot]).wait()

            for b in range(_LOOK):
                start_gather(b, b)

            def body(j):
                for b in range(_NBUF):
                    i = j + b
                    g = i + _LOOK
                    slot_g = (b + _LOOK) % _NBUF

                    @pl.when(jnp.logical_and(g >= _NBUF, g < nch))
                    def _():
                        wait_write(slot_g)

                    @pl.when(g < nch)
                    def _():
                        start_gather(g, slot_g)

                    wait_gather(b)
                    start_write(i, b)

            lax.fori_loop(0, nch // _NBUF,
                          lambda t, _: (body(t * _NBUF), 0)[1], 0)
            for b in range(_NBUF):
                wait_write(b)

        pl.run_scoped(
            run, pltpu.VMEM((_NBUF, chunk, table_hbm.shape[1]), jnp.float32))

    phase(idx_a_hbm, table_a_hbm, out_a_hbm, _CHUNK_A)
    phase(idx_b_hbm, table_b_hbm, out_b_hbm, _CHUNK_B)


def kernel(dom_a_idx, dom_b_idx, table_a, table_b):
    idx_a = dom_a_idx.reshape(_N)
    idx_b = dom_b_idx.reshape(_N)
    out_a, out_b = _gather_kernel(idx_a, idx_b, table_a, table_b)
    return (out_a.reshape(_BATCH, _HIST, _DIM_A),
            out_b.reshape(_BATCH, _HIST, _DIM_B))


# speedup vs baseline: 1.4481x; 1.4481x over previous
import functools

import jax
import jax.numpy as jnp
from jax import lax
from jax.experimental import pallas as pl
from jax.experimental.pallas import tpu as pltpu
from jax.experimental.pallas import tpu_sc as plsc

_DIM_A = 32
_DIM_B = 64
_BATCH, _HIST = 16384, 50
_N = _BATCH * _HIST
_NC, _NS = 2, 16
_NW = _NC * _NS
_PER_W = _N // _NW
_ROWS_W = _BATCH // _NW
_CR_A = 8
_CR_B = 4
_NBUF = 4
_LOOK = 2

_mesh = plsc.VectorSubcoreMesh(core_axis_name="c", subcore_axis_name="s")


@functools.partial(
    pl.kernel,
    mesh=_mesh,
    out_type=[
        jax.ShapeDtypeStruct((_BATCH, _HIST, _DIM_A), jnp.float32),
        jax.ShapeDtypeStruct((_BATCH, _HIST, _DIM_B), jnp.float32),
    ],
    scratch_types=[
        pltpu.VMEM((_PER_W,), jnp.int32),
        pltpu.SemaphoreType.DMA((_NBUF,)),
        pltpu.SemaphoreType.DMA((_NBUF,)),
    ],
    compiler_params=pltpu.CompilerParams(use_tc_tiling_on_sc=False),
)
def _gather_kernel(idx_a_hbm, idx_b_hbm, table_a_hbm, table_b_hbm,
                   out_a_hbm, out_b_hbm, idx_v, sem_g, sem_w):
    wid = lax.axis_index("s") * _NC + lax.axis_index("c")
    base = wid * _PER_W
    row0 = wid * _ROWS_W

    def phase(idx_hbm, table_hbm, out_hbm, cr):
        chunk = cr * _HIST
        nch = _ROWS_W // cr
        dim = table_hbm.shape[1]
        pltpu.sync_copy(idx_hbm.at[pl.ds(base, _PER_W)], idx_v)

        def run(rows):
            def start_gather(i, slot):
                pltpu.make_async_copy(
                    table_hbm.at[idx_v.at[pl.ds(i * chunk, chunk)]],
                    rows.at[slot], sem_g.at[slot]).start()

            def wait_gather(slot):
                pltpu.make_async_copy(
                    table_hbm.at[idx_v.at[pl.ds(0, chunk)]],
                    rows.at[slot], sem_g.at[slot]).wait()

            def start_write(i, slot):
                for r in range(cr):
                    pltpu.make_async_copy(
                        rows.at[slot, pl.ds(r * _HIST, _HIST)],
                        out_hbm.at[row0 + i * cr + r],
                        sem_w.at[slot]).start()

            def wait_write(slot):
                for r in range(cr):
                    pltpu.make_async_copy(
                        rows.at[slot, pl.ds(r * _HIST, _HIST)],
                        out_hbm.at[row0],
                        sem_w.at[slot]).wait()

            for b in range(_LOOK):
                start_gather(b, b)

            def body(j):
                for b in range(_NBUF):
                    i = j + b
                    g = i + _LOOK
                    slot_g = (b + _LOOK) % _NBUF

                    @pl.when(jnp.logical_and(g >= _NBUF, g < nch))
                    def _():
                        wait_write(slot_g)

                    @pl.when(g < nch)
                    def _():
                        start_gather(g, slot_g)

                    wait_gather(b)
                    start_write(i, b)

            lax.fori_loop(0, nch // _NBUF,
                          lambda t, _: (body(t * _NBUF), 0)[1], 0)
            for b in range(_NBUF):
                wait_write(b)

        pl.run_scoped(run, pltpu.VMEM((_NBUF, chunk, dim), jnp.float32))

    phase(idx_a_hbm, table_a_hbm, out_a_hbm, _CR_A)
    phase(idx_b_hbm, table_b_hbm, out_b_hbm, _CR_B)


def kernel(dom_a_idx, dom_b_idx, table_a, table_b):
    idx_a = dom_a_idx.reshape(_N)
    idx_b = dom_b_idx.reshape(_N)
    return tuple(_gather_kernel(idx_a, idx_b, table_a, table_b))


# speedup vs baseline: 1.4498x; 1.0012x over previous
import functools

import jax
import jax.numpy as jnp
from jax import lax
from jax.experimental import pallas as pl
from jax.experimental.pallas import tpu as pltpu
from jax.experimental.pallas import tpu_sc as plsc

_DIM_A = 32
_DIM_B = 64
_BATCH, _HIST = 16384, 50
_N = _BATCH * _HIST
_NC, _NS = 2, 16
_NW = _NC * _NS
_PER_W = _N // _NW
_ROWS_W = _BATCH // _NW
_CR_A = 8
_CR_B = 4
_NBUF = 4
_LOOK = 2

_mesh = plsc.VectorSubcoreMesh(core_axis_name="c", subcore_axis_name="s")


@functools.partial(
    pl.kernel,
    mesh=_mesh,
    out_type=[
        jax.ShapeDtypeStruct((_BATCH, _HIST, _DIM_A), jnp.float32),
        jax.ShapeDtypeStruct((_BATCH, _HIST, _DIM_B), jnp.float32),
    ],
    scratch_types=[
        pltpu.VMEM((_PER_W,), jnp.int32),
        pltpu.SemaphoreType.DMA((_NBUF,)),
        pltpu.SemaphoreType.DMA((_NBUF,)),
    ],
    compiler_params=pltpu.CompilerParams(use_tc_tiling_on_sc=False),
)
def _gather_kernel(idx_a_hbm, idx_b_hbm, table_a_hbm, table_b_hbm,
                   out_a_hbm, out_b_hbm, idx_v, sem_g, sem_w):
    wid = lax.axis_index("s") * _NC + lax.axis_index("c")
    base = wid * _PER_W
    row0 = wid * _ROWS_W

    def phase(idx_hbm, table_hbm, out_hbm, cr):
        chunk = cr * _HIST
        nch = _ROWS_W // cr
        pltpu.sync_copy(idx_hbm.at[pl.ds(base, _PER_W)], idx_v)

        def run(rows):
            def start_gather(i, slot):
                pltpu.make_async_copy(
                    table_hbm.at[idx_v.at[pl.ds(i * chunk, chunk)]],
                    rows.at[slot], sem_g.at[slot]).start()

            def wait_gather(slot):
                pltpu.make_async_copy(
                    table_hbm.at[idx_v.at[pl.ds(0, chunk)]],
                    rows.at[slot], sem_g.at[slot]).wait()

            def start_write(i, slot):
                for r in range(cr):
                    pltpu.make_async_copy(
                        rows.at[slot, pl.ds(r * _HIST, _HIST)],
                        out_hbm.at[row0 + i * cr + r],
                        sem_w.at[slot]).start()

            def wait_write(slot):
                for r in range(cr):
                    pltpu.make_async_copy(
                        rows.at[slot, pl.ds(r * _HIST, _HIST)],
                        out_hbm.at[row0],
                        sem_w.at[slot]).wait()

            for b in range(_LOOK):
                start_gather(b, b)

            def body(j):
                for b in range(_NBUF):
                    i = j + b
                    g = i + _LOOK
                    slot_g = (b + _LOOK) % _NBUF

                    @pl.when(jnp.logical_and(g >= _NBUF, g < nch))
                    def _():
                        wait_write(slot_g)

                    @pl.when(g < nch)
                    def _():
                        start_gather(g, slot_g)

                    wait_gather(b)
                    start_write(i, b)

            lax.fori_loop(0, nch // _NBUF,
                          lambda t, _: (body(t * _NBUF), 0)[1], 0)
            for b in range(_NBUF):
                wait_write(b)

        pl.run_scoped(run, pltpu.VMEM((_NBUF, chunk, table_hbm.shape[1]),
                                      jnp.float32))

    phase(idx_a_hbm, table_a_hbm, out_a_hbm, _CR_A)
    phase(idx_b_hbm, table_b_hbm, out_b_hbm, _CR_B)


def kernel(dom_a_idx, dom_b_idx, table_a, table_b):
    idx_a = dom_a_idx.reshape(_N)
    idx_b = dom_b_idx.reshape(_N)
    return tuple(_gather_kernel(idx_a, idx_b, table_a, table_b))
